# baseline (speedup 1.0000x reference)
import functools

import jax
import jax.numpy as jnp
from jax import lax
from jax.experimental import pallas as pl
from jax.experimental.pallas import tpu as pltpu
from jax.experimental.pallas import tpu_sc as plsc

N = 10000
E = 320000
D_IN = 128
D_H = 256
N_CLS = 64

NC = 2
NS = 16
NW = NC * NS

PAD = 16
C = 125
EPW = E // NW
K = EPW // C

ROWS_A = 640
ROWS_B = N - ROWS_A * (NS - 1)
ZR = 16


def _make_agg(Dp, spmem_feat=False):
    mesh = plsc.VectorSubcoreMesh(core_axis_name="c", subcore_axis_name="s",
                                  num_cores=NC, num_subcores=NS)

    scratch = [
        pltpu.VMEM_SHARED((N, Dp), jnp.float32),
        pltpu.VMEM((2, C), jnp.int32),
        pltpu.VMEM((2, C), jnp.int32),
        pltpu.VMEM((C, Dp), jnp.float32),
        pltpu.VMEM((C, Dp), jnp.float32),
        pltpu.VMEM((ZR, Dp), jnp.float32),
        pltpu.SemaphoreType.DMA,
        pltpu.SemaphoreType.DMA,
        pltpu.SemaphoreType.DMA,
        pltpu.SemaphoreType.DMA,
    ]
    if spmem_feat:
        scratch.append(pltpu.VMEM_SHARED((N, Dp), jnp.float32))

    @functools.partial(
        pl.kernel,
        out_type=jax.ShapeDtypeStruct((NC, N, Dp), jnp.float32),
        mesh=mesh,
        scratch_types=scratch,
        compiler_params=pltpu.CompilerParams(use_tc_tiling_on_sc=False),
    )
    def agg(feat, eidx, sums_out, acc, idx_a, idx_b,
            rows_a, rows_b, zrow, sem_a, sem_b, sem_ia, sem_ib,
            *maybe_fsp):
        cid = lax.axis_index("c")
        sid = lax.axis_index("s")
        wid = cid * NS + sid
        gsrc = maybe_fsp[0] if spmem_feat else feat

        z16 = jnp.zeros((16,), jnp.float32)

        def zfill(i, _):
            r = i // (Dp // 16)
            col = (i % (Dp // 16)) * 16
            zrow[r, pl.ds(col, 16)] = z16
            return 0

        lax.fori_loop(0, ZR * (Dp // 16), zfill, 0)

        @pl.when(sid < NS - 1)
        def _():
            def zb(k, _):
                pltpu.sync_copy(zrow, acc.at[pl.ds(sid * ROWS_A + k * ZR, ZR)])
                return 0
            lax.fori_loop(0, ROWS_A // ZR, zb, 0)

        @pl.when(sid == NS - 1)
        def _():
            def zb(k, _):
                pltpu.sync_copy(zrow, acc.at[pl.ds((NS - 1) * ROWS_A + k * ZR, ZR)])
                return 0
            lax.fori_loop(0, ROWS_B // ZR, zb, 0)

        if spmem_feat:
            @pl.when(sid < NS - 1)
            def _():
                base = sid * ROWS_A
                pltpu.sync_copy(feat.at[pl.ds(base, ROWS_A)],
                                gsrc.at[pl.ds(base, ROWS_A)])

            @pl.when(sid == NS - 1)
            def _():
                base = (NS - 1) * ROWS_A
                pltpu.sync_copy(feat.at[pl.ds(base, ROWS_B)],
                                gsrc.at[pl.ds(base, ROWS_B)])

        plsc.subcore_barrier()

        pltpu.sync_copy(eidx.at[wid, 0], idx_a)
        pltpu.async_copy(gsrc.at[idx_a.at[0]], rows_a, sem_a)
        pltpu.async_copy(eidx.at[wid, 1], idx_b, sem_ib)

        def ebody(i, _):
            j = 2 * i
            pltpu.make_async_copy(gsrc.at[idx_a.at[0]], rows_a, sem_a).wait()
            pltpu.make_async_copy(eidx.at[wid, j + 1], idx_b, sem_ib).wait()
            pltpu.async_copy(gsrc.at[idx_b.at[0]], rows_b, sem_b)
            pltpu.sync_copy(rows_a, acc.at[idx_a.at[1]], add=True)

            @pl.when(j + 2 < K)
            def _():
                pltpu.async_copy(eidx.at[wid, j + 2], idx_a, sem_ia)

            pltpu.make_async_copy(gsrc.at[idx_b.at[0]], rows_b, sem_b).wait()

            @pl.when(j + 2 < K)
            def _():
                pltpu.make_async_copy(eidx.at[wid, j + 2], idx_a, sem_ia).wait()
                pltpu.async_copy(gsrc.at[idx_a.at[0]], rows_a, sem_a)

            pltpu.sync_copy(rows_b, acc.at[idx_b.at[1]], add=True)

            @pl.when(j + 3 < K)
            def _():
                pltpu.async_copy(eidx.at[wid, j + 3], idx_b, sem_ib)
            return 0

        lax.fori_loop(0, K // 2, ebody, 0)

        plsc.subcore_barrier()

        @pl.when(sid < NS - 1)
        def _():
            base = sid * ROWS_A
            pltpu.sync_copy(acc.at[pl.ds(base, ROWS_A)],
                            sums_out.at[cid, pl.ds(base, ROWS_A)])

        @pl.when(sid == NS - 1)
        def _():
            base = (NS - 1) * ROWS_A
            pltpu.sync_copy(acc.at[pl.ds(base, ROWS_B)],
                            sums_out.at[cid, pl.ds(base, ROWS_B)])

    return agg


_agg_l0 = _make_agg(D_IN + PAD)
_agg_l1 = _make_agg(N_CLS + PAD, spmem_feat=True)

_R1 = 1000
_R2 = 1000


def _tc_fuse(p0, W0, b0, W1):
    D0 = D_IN + PAD

    def body(p_ref, w0_ref, b0_ref, w1_ref, z_ref):
        a = p_ref[0] + p_ref[1]
        c = a[:, D_IN:D_IN + 1]
        inv = 1.0 / jnp.maximum(c, 1.0)
        mask = (c > 0.0).astype(jnp.float32)
        feats = a[:, :D_IN] * inv
        h = jnp.dot(feats, w0_ref[...], preferred_element_type=jnp.float32)
        h = jnp.maximum(h + b0_ref[...] * mask, 0.0)
        z = jnp.dot(h, w1_ref[...], preferred_element_type=jnp.float32)
        z_ref[...] = jnp.concatenate(
            [z, jnp.ones((_R1, 1), jnp.float32),
             jnp.zeros((_R1, PAD - 1), jnp.float32)], axis=1)

    return pl.pallas_call(
        body,
        grid=(N // _R1,),
        in_specs=[
            pl.BlockSpec((NC, _R1, D0), lambda i: (0, i, 0)),
            pl.BlockSpec((D_IN, D_H), lambda i: (0, 0)),
            pl.BlockSpec((1, D_H), lambda i: (0, 0)),
            pl.BlockSpec((D_H, N_CLS), lambda i: (0, 0)),
        ],
        out_specs=pl.BlockSpec((_R1, N_CLS + PAD), lambda i: (i, 0)),
        out_shape=jax.ShapeDtypeStruct((N, N_CLS + PAD), jnp.float32),
    )(p0, W0, b0, W1)


def _tc_final(p1, b1):
    D1 = N_CLS + PAD

    def body(p_ref, b_ref, o_ref):
        s = p_ref[0] + p_ref[1]
        c = s[:, N_CLS:N_CLS + 1]
        inv = 1.0 / jnp.maximum(c, 1.0)
        mask = (c > 0.0).astype(jnp.float32)
        o_ref[...] = s[:, :N_CLS] * inv + b_ref[...] * mask

    return pl.pallas_call(
        body,
        grid=(N // _R2,),
        in_specs=[
            pl.BlockSpec((NC, _R2, D1), lambda i: (0, i, 0)),
            pl.BlockSpec((1, N_CLS), lambda i: (0, 0)),
        ],
        out_specs=pl.BlockSpec((_R2, N_CLS), lambda i: (i, 0)),
        out_shape=jax.ShapeDtypeStruct((N, N_CLS), jnp.float32),
    )(p1, b1)


def kernel(x, edge_index0, edge_index1, W0, b0, W1, b1):
    x_pad = jnp.concatenate(
        [x, jnp.ones((N, 1), jnp.float32),
         jnp.zeros((N, PAD - 1), jnp.float32)], axis=1)
    eidx0 = jnp.stack([edge_index0[0].reshape(NW, K, C),
                       edge_index0[1].reshape(NW, K, C)], axis=2)
    eidx1 = jnp.stack([edge_index1[0].reshape(NW, K, C),
                       edge_index1[1].reshape(NW, K, C)], axis=2)

    p0 = _agg_l0(x_pad, eidx0)
    z_pad = _tc_fuse(p0, W0, b0.reshape(1, D_H), W1)
    p1 = _agg_l1(z_pad, eidx1)
    return _tc_final(p1, b1.reshape(1, N_CLS))


# speedup vs baseline: 1.0626x; 1.0626x over previous
import functools

import jax
import jax.numpy as jnp
from jax import lax
from jax.experimental import pallas as pl
from jax.experimental.pallas import tpu as pltpu
from jax.experimental.pallas import tpu_sc as plsc

N = 10000
E = 320000
D_IN = 128
D_H = 256
N_CLS = 64

NC = 2
NS = 16
NW = NC * NS

PAD = 16
C = 125
EPW = E // NW
K = EPW // C

ROWS_A = 640
ROWS_B = N - ROWS_A * (NS - 1)
ZR = 16


def _make_agg(Dp, spmem_feat=False):
    mesh = plsc.VectorSubcoreMesh(core_axis_name="c", subcore_axis_name="s",
                                  num_cores=NC, num_subcores=NS)

    scratch = [
        pltpu.VMEM_SHARED((N, Dp), jnp.float32),
        pltpu.VMEM((2, C), jnp.int32),
        pltpu.VMEM((2, C), jnp.int32),
        pltpu.VMEM((C, Dp), jnp.float32),
        pltpu.VMEM((C, Dp), jnp.float32),
        pltpu.VMEM((ZR, Dp), jnp.float32),
        pltpu.SemaphoreType.DMA,
        pltpu.SemaphoreType.DMA,
        pltpu.SemaphoreType.DMA,
        pltpu.SemaphoreType.DMA,
    ]
    if spmem_feat:
        scratch.append(pltpu.VMEM_SHARED((N, Dp), jnp.float32))

    @functools.partial(
        pl.kernel,
        out_type=jax.ShapeDtypeStruct((NC, N, Dp), jnp.float32),
        mesh=mesh,
        scratch_types=scratch,
        compiler_params=pltpu.CompilerParams(use_tc_tiling_on_sc=False),
    )
    def agg(feat, eidx, sums_out, acc, idx_a, idx_b,
            rows_a, rows_b, zrow, sem_a, sem_b, sem_ia, sem_ib,
            *maybe_fsp):
        cid = lax.axis_index("c")
        sid = lax.axis_index("s")
        wid = cid * NS + sid
        gsrc = maybe_fsp[0] if spmem_feat else feat

        z16 = jnp.zeros((16,), jnp.float32)

        def zfill(i, _):
            r = i // (Dp // 16)
            col = (i % (Dp // 16)) * 16
            zrow[r, pl.ds(col, 16)] = z16
            return 0

        lax.fori_loop(0, ZR * (Dp // 16), zfill, 0)

        @pl.when(sid < NS - 1)
        def _():
            def zb(k, _):
                pltpu.sync_copy(zrow, acc.at[pl.ds(sid * ROWS_A + k * ZR, ZR)])
                return 0
            lax.fori_loop(0, ROWS_A // ZR, zb, 0)

        @pl.when(sid == NS - 1)
        def _():
            def zb(k, _):
                pltpu.sync_copy(zrow, acc.at[pl.ds((NS - 1) * ROWS_A + k * ZR, ZR)])
                return 0
            lax.fori_loop(0, ROWS_B // ZR, zb, 0)

        if spmem_feat:
            @pl.when(sid < NS - 1)
            def _():
                base = sid * ROWS_A
                pltpu.sync_copy(feat.at[pl.ds(base, ROWS_A)],
                                gsrc.at[pl.ds(base, ROWS_A)])

            @pl.when(sid == NS - 1)
            def _():
                base = (NS - 1) * ROWS_A
                pltpu.sync_copy(feat.at[pl.ds(base, ROWS_B)],
                                gsrc.at[pl.ds(base, ROWS_B)])

        plsc.subcore_barrier()

        pltpu.sync_copy(eidx.at[wid, 0], idx_a)
        pltpu.async_copy(gsrc.at[idx_a.at[0]], rows_a, sem_a)
        pltpu.async_copy(eidx.at[wid, 1], idx_b, sem_ib)

        def ebody(i, _):
            j = 2 * i
            pltpu.make_async_copy(gsrc.at[idx_a.at[0]], rows_a, sem_a).wait()
            pltpu.make_async_copy(eidx.at[wid, j + 1], idx_b, sem_ib).wait()
            pltpu.async_copy(gsrc.at[idx_b.at[0]], rows_b, sem_b)
            pltpu.sync_copy(rows_a, acc.at[idx_a.at[1]], add=True)

            @pl.when(j + 2 < K)
            def _():
                pltpu.async_copy(eidx.at[wid, j + 2], idx_a, sem_ia)

            pltpu.make_async_copy(gsrc.at[idx_b.at[0]], rows_b, sem_b).wait()

            @pl.when(j + 2 < K)
            def _():
                pltpu.make_async_copy(eidx.at[wid, j + 2], idx_a, sem_ia).wait()
                pltpu.async_copy(gsrc.at[idx_a.at[0]], rows_a, sem_a)

            pltpu.sync_copy(rows_b, acc.at[idx_b.at[1]], add=True)

            @pl.when(j + 3 < K)
            def _():
                pltpu.async_copy(eidx.at[wid, j + 3], idx_b, sem_ib)
            return 0

        lax.fori_loop(0, K // 2, ebody, 0)

        plsc.subcore_barrier()

        @pl.when(sid < NS - 1)
        def _():
            base = sid * ROWS_A
            pltpu.sync_copy(acc.at[pl.ds(base, ROWS_A)],
                            sums_out.at[cid, pl.ds(base, ROWS_A)])

        @pl.when(sid == NS - 1)
        def _():
            base = (NS - 1) * ROWS_A
            pltpu.sync_copy(acc.at[pl.ds(base, ROWS_B)],
                            sums_out.at[cid, pl.ds(base, ROWS_B)])

    return agg


_agg_l0 = _make_agg(D_IN + PAD)
_agg_l1 = _make_agg(N_CLS + PAD)

_R1 = 1000
_R2 = 1000


def _tc_fuse(p0, W0, b0, W1):
    D0 = D_IN + PAD

    def body(p_ref, w0_ref, b0_ref, w1_ref, z_ref):
        a = p_ref[0] + p_ref[1]
        c = a[:, D_IN:D_IN + 1]
        inv = 1.0 / jnp.maximum(c, 1.0)
        mask = (c > 0.0).astype(jnp.float32)
        feats = a[:, :D_IN] * inv
        h = jnp.dot(feats, w0_ref[...], preferred_element_type=jnp.float32)
        h = jnp.maximum(h + b0_ref[...] * mask, 0.0)
        z = jnp.dot(h, w1_ref[...], preferred_element_type=jnp.float32)
        z_ref[...] = jnp.concatenate(
            [z, jnp.ones((_R1, 1), jnp.float32),
             jnp.zeros((_R1, PAD - 1), jnp.float32)], axis=1)

    return pl.pallas_call(
        body,
        grid=(N // _R1,),
        in_specs=[
            pl.BlockSpec((NC, _R1, D0), lambda i: (0, i, 0)),
            pl.BlockSpec((D_IN, D_H), lambda i: (0, 0)),
            pl.BlockSpec((1, D_H), lambda i: (0, 0)),
            pl.BlockSpec((D_H, N_CLS), lambda i: (0, 0)),
        ],
        out_specs=pl.BlockSpec((_R1, N_CLS + PAD), lambda i: (i, 0)),
        out_shape=jax.ShapeDtypeStruct((N, N_CLS + PAD), jnp.float32),
    )(p0, W0, b0, W1)


def _tc_final(p1, b1):
    D1 = N_CLS + PAD

    def body(p_ref, b_ref, o_ref):
        s = p_ref[0] + p_ref[1]
        c = s[:, N_CLS:N_CLS + 1]
        inv = 1.0 / jnp.maximum(c, 1.0)
        mask = (c > 0.0).astype(jnp.float32)
        o_ref[...] = s[:, :N_CLS] * inv + b_ref[...] * mask

    return pl.pallas_call(
        body,
        grid=(N // _R2,),
        in_specs=[
            pl.BlockSpec((NC, _R2, D1), lambda i: (0, i, 0)),
            pl.BlockSpec((1, N_CLS), lambda i: (0, 0)),
        ],
        out_specs=pl.BlockSpec((_R2, N_CLS), lambda i: (i, 0)),
        out_shape=jax.ShapeDtypeStruct((N, N_CLS), jnp.float32),
    )(p1, b1)


def kernel(x, edge_index0, edge_index1, W0, b0, W1, b1):
    x_pad = jnp.concatenate(
        [x, jnp.ones((N, 1), jnp.float32),
         jnp.zeros((N, PAD - 1), jnp.float32)], axis=1)
    eidx0 = jnp.stack([edge_index0[0].reshape(NW, K, C),
                       edge_index0[1].reshape(NW, K, C)], axis=2)
    eidx1 = jnp.stack([edge_index1[0].reshape(NW, K, C),
                       edge_index1[1].reshape(NW, K, C)], axis=2)

    p0 = _agg_l0(x_pad, eidx0)
    z_pad = _tc_fuse(p0, W0, b0.reshape(1, D_H), W1)
    p1 = _agg_l1(z_pad, eidx1)
    return _tc_final(p1, b1.reshape(1, N_CLS))


# speedup vs baseline: 1.2036x; 1.1326x over previous
import functools

import jax
import jax.numpy as jnp
from jax import lax
from jax.experimental import pallas as pl
from jax.experimental.pallas import tpu as pltpu
from jax.experimental.pallas import tpu_sc as plsc

N = 10000
E = 320000
D_IN = 128
D_H = 256
N_CLS = 64

NC = 2
NS = 16
NW = NC * NS

CW = 16
C = 125
EPW = E // NW
K = EPW // C

ROWS_A = 640
ROWS_B = N - ROWS_A * (NS - 1)
ZR = 16


def _make_agg(D):
    mesh = plsc.VectorSubcoreMesh(core_axis_name="c", subcore_axis_name="s",
                                  num_cores=NC, num_subcores=NS)

    @functools.partial(
        pl.kernel,
        out_type=(jax.ShapeDtypeStruct((NC, N, D), jnp.float32),
                  jax.ShapeDtypeStruct((NC, N, CW), jnp.float32)),
        mesh=mesh,
        scratch_types=[
            pltpu.VMEM_SHARED((N, D), jnp.float32),
            pltpu.VMEM_SHARED((N, CW), jnp.float32),
            pltpu.VMEM((2, C), jnp.int32),
            pltpu.VMEM((2, C), jnp.int32),
            pltpu.VMEM((C, D), jnp.float32),
            pltpu.VMEM((C, D), jnp.float32),
            pltpu.VMEM((C, CW), jnp.float32),
            pltpu.VMEM((ZR, D), jnp.float32),
            pltpu.VMEM((ZR, CW), jnp.float32),
            pltpu.SemaphoreType.DMA,
            pltpu.SemaphoreType.DMA,
            pltpu.SemaphoreType.DMA,
            pltpu.SemaphoreType.DMA,
        ],
        compiler_params=pltpu.CompilerParams(use_tc_tiling_on_sc=False),
    )
    def agg(feat, src2, dst2, sums_out, cnt_out, acc, cnt, idx_a, idx_b,
            rows_a, rows_b, ones_blk, zrow, zcnt, sem_a, sem_b, sem_ia, sem_ib):
        cid = lax.axis_index("c")
        sid = lax.axis_index("s")
        wid = cid * NS + sid

        z16 = jnp.zeros((16,), jnp.float32)
        o16 = jnp.ones((16,), jnp.float32)

        def zfill(i, _):
            r = i // (D // 16)
            col = (i % (D // 16)) * 16
            zrow[r, pl.ds(col, 16)] = z16
            return 0

        lax.fori_loop(0, ZR * (D // 16), zfill, 0)

        def zcfill(i, _):
            zcnt[i, pl.ds(0, CW)] = z16[:CW]
            return 0

        lax.fori_loop(0, ZR, zcfill, 0)

        def ofill(i, _):
            ones_blk[i, pl.ds(0, CW)] = o16[:CW]
            return 0

        lax.fori_loop(0, C, ofill, 0)

        @pl.when(sid < NS - 1)
        def _():
            def zb(k, _):
                pltpu.sync_copy(zrow, acc.at[pl.ds(sid * ROWS_A + k * ZR, ZR)])
                pltpu.sync_copy(zcnt, cnt.at[pl.ds(sid * ROWS_A + k * ZR, ZR)])
                return 0
            lax.fori_loop(0, ROWS_A // ZR, zb, 0)

        @pl.when(sid == NS - 1)
        def _():
            def zb(k, _):
                base = (NS - 1) * ROWS_A + k * ZR
                pltpu.sync_copy(zrow, acc.at[pl.ds(base, ZR)])
                pltpu.sync_copy(zcnt, cnt.at[pl.ds(base, ZR)])
                return 0
            lax.fori_loop(0, ROWS_B // ZR, zb, 0)

        plsc.subcore_barrier()

        def idx_start(k, buf, sem):
            pltpu.async_copy(src2.at[wid, k], buf.at[0], sem)
            pltpu.async_copy(dst2.at[wid, k], buf.at[1], sem)

        def idx_wait(k, buf, sem):
            pltpu.make_async_copy(src2.at[wid, k], buf.at[0], sem).wait()
            pltpu.make_async_copy(dst2.at[wid, k], buf.at[1], sem).wait()

        def scat(rows, buf):
            pltpu.sync_copy(rows, acc.at[buf.at[1]], add=True)
            pltpu.sync_copy(ones_blk, cnt.at[buf.at[1]], add=True)

        idx_start(0, idx_a, sem_ia)
        idx_wait(0, idx_a, sem_ia)
        pltpu.async_copy(feat.at[idx_a.at[0]], rows_a, sem_a)
        idx_start(1, idx_b, sem_ib)

        def ebody(i, _):
            j = 2 * i
            pltpu.make_async_copy(feat.at[idx_a.at[0]], rows_a, sem_a).wait()
            idx_wait(j + 1, idx_b, sem_ib)
            pltpu.async_copy(feat.at[idx_b.at[0]], rows_b, sem_b)
            scat(rows_a, idx_a)

            @pl.when(j + 2 < K)
            def _():
                idx_start(j + 2, idx_a, sem_ia)

            pltpu.make_async_copy(feat.at[idx_b.at[0]], rows_b, sem_b).wait()

            @pl.when(j + 2 < K)
            def _():
                idx_wait(j + 2, idx_a, sem_ia)
                pltpu.async_copy(feat.at[idx_a.at[0]], rows_a, sem_a)

            scat(rows_b, idx_b)

            @pl.when(j + 3 < K)
            def _():
                idx_start(j + 3, idx_b, sem_ib)
            return 0

        lax.fori_loop(0, K // 2, ebody, 0)

        plsc.subcore_barrier()

        @pl.when(sid < NS - 1)
        def _():
            base = sid * ROWS_A
            pltpu.sync_copy(acc.at[pl.ds(base, ROWS_A)],
                            sums_out.at[cid, pl.ds(base, ROWS_A)])
            pltpu.sync_copy(cnt.at[pl.ds(base, ROWS_A)],
                            cnt_out.at[cid, pl.ds(base, ROWS_A)])

        @pl.when(sid == NS - 1)
        def _():
            base = (NS - 1) * ROWS_A
            pltpu.sync_copy(acc.at[pl.ds(base, ROWS_B)],
                            sums_out.at[cid, pl.ds(base, ROWS_B)])
            pltpu.sync_copy(cnt.at[pl.ds(base, ROWS_B)],
                            cnt_out.at[cid, pl.ds(base, ROWS_B)])

    return agg


_agg_l0 = _make_agg(D_IN)
_agg_l1 = _make_agg(N_CLS)

_R1 = 1000
_R2 = 1000


def _tc_fuse(p0, c0, W0, b0, W1):
    def body(p_ref, c_ref, w0_ref, b0_ref, w1_ref, z_ref):
        a = p_ref[0] + p_ref[1]
        c16 = c_ref[0] + c_ref[1]
        c = c16[:, 0:1]
        inv = 1.0 / jnp.maximum(c, 1.0)
        mask = (c > 0.0).astype(jnp.float32)
        feats = a * inv
        h = jnp.dot(feats, w0_ref[...], preferred_element_type=jnp.float32)
        h = jnp.maximum(h + b0_ref[...] * mask, 0.0)
        z_ref[...] = jnp.dot(h, w1_ref[...], preferred_element_type=jnp.float32)

    return pl.pallas_call(
        body,
        grid=(N // _R1,),
        in_specs=[
            pl.BlockSpec((NC, _R1, D_IN), lambda i: (0, i, 0)),
            pl.BlockSpec((NC, _R1, CW), lambda i: (0, i, 0)),
            pl.BlockSpec((D_IN, D_H), lambda i: (0, 0)),
            pl.BlockSpec((1, D_H), lambda i: (0, 0)),
            pl.BlockSpec((D_H, N_CLS), lambda i: (0, 0)),
        ],
        out_specs=pl.BlockSpec((_R1, N_CLS), lambda i: (i, 0)),
        out_shape=jax.ShapeDtypeStruct((N, N_CLS), jnp.float32),
    )(p0, c0, W0, b0, W1)


def _tc_final(p1, c1, b1):
    def body(p_ref, c_ref, b_ref, o_ref):
        s = p_ref[0] + p_ref[1]
        c16 = c_ref[0] + c_ref[1]
        c = c16[:, 0:1]
        inv = 1.0 / jnp.maximum(c, 1.0)
        mask = (c > 0.0).astype(jnp.float32)
        o_ref[...] = s * inv + b_ref[...] * mask

    return pl.pallas_call(
        body,
        grid=(N // _R2,),
        in_specs=[
            pl.BlockSpec((NC, _R2, N_CLS), lambda i: (0, i, 0)),
            pl.BlockSpec((NC, _R2, CW), lambda i: (0, i, 0)),
            pl.BlockSpec((1, N_CLS), lambda i: (0, 0)),
        ],
        out_specs=pl.BlockSpec((_R2, N_CLS), lambda i: (i, 0)),
        out_shape=jax.ShapeDtypeStruct((N, N_CLS), jnp.float32),
    )(p1, c1, b1)


def kernel(x, edge_index0, edge_index1, W0, b0, W1, b1):
    src0 = edge_index0[0].reshape(NW, K, C)
    dst0 = edge_index0[1].reshape(NW, K, C)
    src1 = edge_index1[0].reshape(NW, K, C)
    dst1 = edge_index1[1].reshape(NW, K, C)

    p0, c0 = _agg_l0(x, src0, dst0)
    z = _tc_fuse(p0, c0, W0, b0.reshape(1, D_H), W1)
    p1, c1 = _agg_l1(z, src1, dst1)
    return _tc_final(p1, c1, b1.reshape(1, N_CLS))
